# baseline (speedup 1.0000x reference)
import functools

import jax
import jax.numpy as jnp
from jax import lax
from jax.experimental import pallas as pl
from jax.experimental.pallas import tpu as pltpu
from jax.experimental.pallas import tpu_sc as plsc

N = 10000
NP = 10112
E = 160000
B = 128
NB = 80
ET = NB * B
EP = 16 * ET
D = 256
H = 512
F = 128
C1 = D // F
C2 = H // F
G = 64
NC, NS = 2, 16
RPT = NP // NS
BN = 1000
GRID = N // BN


def _sc_mesh():
    return plsc.VectorSubcoreMesh(core_axis_name="c", subcore_axis_name="s")


def _sc_degree(cols2d, ones_src, zerosf):
    @functools.partial(
        pl.kernel,
        mesh=_sc_mesh(),
        out_type=jax.ShapeDtypeStruct((NC, NP, F), jnp.float32),
        scratch_types=[
            pltpu.VMEM((NB, B), jnp.int32),
            pltpu.VMEM((B, F), jnp.float32),
            pltpu.VMEM_SHARED((NP, F), jnp.float32),
        ],
    )
    def k(cols_hbm, ones_hbm, z_hbm, out_hbm, cols_v, ones_v, acc):
        c = lax.axis_index("c")
        s = lax.axis_index("s")
        pltpu.sync_copy(cols_hbm.at[s], cols_v)
        pltpu.sync_copy(ones_hbm, ones_v)
        pltpu.sync_copy(z_hbm.at[pl.ds(s * RPT, RPT)], acc.at[pl.ds(s * RPT, RPT)])
        plsc.subcore_barrier()

        def body(j, carry):
            pltpu.sync_copy(ones_v, acc.at[cols_v.at[j]], add=True)
            return carry

        lax.fori_loop(c * (NB // 2), (c + 1) * (NB // 2), body, 0)
        plsc.subcore_barrier()
        pltpu.sync_copy(acc.at[pl.ds(s * RPT, RPT)], out_hbm.at[c, pl.ds(s * RPT, RPT)])

    return k(cols2d, ones_src, zerosf)


def _sc_prop(C, u_flat, rows_off, cols2d, zerosf):
    cpc = C // NC

    @functools.partial(
        pl.kernel,
        mesh=_sc_mesh(),
        out_type=jax.ShapeDtypeStruct((C, NP, F), jnp.float32),
        scratch_types=[
            pltpu.VMEM((NB, B), jnp.int32),
            pltpu.VMEM((NB, B), jnp.int32),
            pltpu.VMEM((B, F), jnp.float32),
            pltpu.VMEM_SHARED((NP, F), jnp.float32),
            pltpu.SemaphoreType.DMA,
        ],
    )
    def k(u_hbm, rows_hbm, cols_hbm, z_hbm, out_hbm, gidx, cols_v, gbuf, acc, sem):
        c = lax.axis_index("c")
        s = lax.axis_index("s")
        pltpu.sync_copy(cols_hbm.at[s], cols_v)
        for ci in range(cpc):
            q = c + NC * ci
            pltpu.sync_copy(rows_hbm.at[q, s], gidx)
            pltpu.sync_copy(z_hbm.at[pl.ds(s * RPT, RPT)], acc.at[pl.ds(s * RPT, RPT)])
            plsc.subcore_barrier()

            def body(j, carry):
                pltpu.async_copy(u_hbm.at[gidx.at[j]], gbuf, sem).wait()
                pltpu.sync_copy(gbuf, acc.at[cols_v.at[j]], add=True)
                return carry

            lax.fori_loop(0, NB, body, 0)
            plsc.subcore_barrier()
            pltpu.sync_copy(acc.at[pl.ds(s * RPT, RPT)], out_hbm.at[q, pl.ds(s * RPT, RPT)])

    return k(u_flat, rows_off, cols2d, zerosf)


def _prelu(v):
    return jnp.where(v >= 0, v, 0.25 * v)


def _prep_call(degacc, x):
    def body(deg_ref, x_ref, dinv_ref, d2_ref, u0_ref):
        deg = deg_ref[0, :, 0:1] + deg_ref[1, :, 0:1]
        dinv = jnp.where(deg > 0, lax.rsqrt(deg), 0.0)
        dinv_ref[...] = dinv
        d2_ref[...] = dinv * dinv
        for q in range(C1):
            u0_ref[q] = x_ref[:, q * F:(q + 1) * F] * dinv

    return pl.pallas_call(
        body,
        grid=(GRID,),
        in_specs=[
            pl.BlockSpec((NC, BN, F), lambda i: (0, i, 0)),
            pl.BlockSpec((BN, D), lambda i: (i, 0)),
        ],
        out_specs=[
            pl.BlockSpec((BN, 1), lambda i: (i, 0)),
            pl.BlockSpec((BN, 1), lambda i: (i, 0)),
            pl.BlockSpec((C1, BN, F), lambda i: (0, i, 0)),
        ],
        out_shape=[
            jax.ShapeDtypeStruct((N, 1), jnp.float32),
            jax.ShapeDtypeStruct((N, 1), jnp.float32),
            jax.ShapeDtypeStruct((C1, NP, F), jnp.float32),
        ],
    )(degacc, x)


def _scale_call(C, p, d2):
    def body(p_ref, d2_ref, u_ref):
        u_ref[0] = p_ref[0] * d2_ref[...]

    return pl.pallas_call(
        body,
        grid=(C, GRID),
        in_specs=[
            pl.BlockSpec((1, BN, F), lambda c, i: (c, i, 0)),
            pl.BlockSpec((BN, 1), lambda c, i: (i, 0)),
        ],
        out_specs=pl.BlockSpec((1, BN, F), lambda c, i: (c, i, 0)),
        out_shape=jax.ShapeDtypeStruct((C, NP, F), jnp.float32),
    )(p, d2)


def _layer1_call(x, p1, p2, p3, dinv, W0, b0r):
    def body(x_ref, p1_ref, p2_ref, p3_ref, dinv_ref, w_ref, b_ref, out_ref, u_ref):
        dinv = dinv_ref[...]
        acc = jnp.dot(x_ref[...], w_ref[0], preferred_element_type=jnp.float32)
        for k, p_ref in ((1, p1_ref), (2, p2_ref), (3, p3_ref)):
            for q in range(C1):
                acc += jnp.dot(p_ref[q] * dinv, w_ref[k, q * F:(q + 1) * F, :],
                               preferred_element_type=jnp.float32)
        o = _prelu(acc + b_ref[...])
        out_ref[...] = o
        for q in range(C2):
            u_ref[q] = o[:, q * F:(q + 1) * F] * dinv

    pspec = pl.BlockSpec((C1, BN, F), lambda i: (0, i, 0))
    return pl.pallas_call(
        body,
        grid=(GRID,),
        in_specs=[
            pl.BlockSpec((BN, D), lambda i: (i, 0)),
            pspec, pspec, pspec,
            pl.BlockSpec((BN, 1), lambda i: (i, 0)),
            pl.BlockSpec((4, D, H), lambda i: (0, 0, 0)),
            pl.BlockSpec((1, H), lambda i: (0, 0)),
        ],
        out_specs=[
            pl.BlockSpec((BN, H), lambda i: (i, 0)),
            pl.BlockSpec((C2, BN, F), lambda i: (0, i, 0)),
        ],
        out_shape=[
            jax.ShapeDtypeStruct((N, H), jnp.float32),
            jax.ShapeDtypeStruct((C2, NP, F), jnp.float32),
        ],
    )(x, p1, p2, p3, dinv, W0, b0r)


def _layer2_call(h, p1, p2, p3, dinv, batch2, W1, b1r, Wout):
    def body(h_ref, p1_ref, p2_ref, p3_ref, dinv_ref, b_ref, w_ref, bias_ref,
             wout_ref, out_ref):
        i = pl.program_id(0)
        dinv = dinv_ref[...]
        acc = jnp.dot(h_ref[...], w_ref[0], preferred_element_type=jnp.float32)
        for k, p_ref in ((1, p1_ref), (2, p2_ref), (3, p3_ref)):
            for q in range(C2):
                acc += jnp.dot(p_ref[q] * dinv, w_ref[k, q * F:(q + 1) * F, :],
                               preferred_element_type=jnp.float32)
        o = _prelu(acc + bias_ref[...])
        z = jnp.dot(o, wout_ref[...], preferred_element_type=jnp.float32)
        gids = lax.broadcasted_iota(jnp.int32, (BN, G), 1)
        mask = (b_ref[...] == gids).astype(jnp.float32)
        pool = jnp.sum(mask * z, axis=0, keepdims=True)

        @pl.when(i == 0)
        def _():
            out_ref[...] = jnp.zeros_like(out_ref)

        out_ref[...] += pool

    pspec = pl.BlockSpec((C2, BN, F), lambda i: (0, i, 0))
    return pl.pallas_call(
        body,
        grid=(GRID,),
        in_specs=[
            pl.BlockSpec((BN, H), lambda i: (i, 0)),
            pspec, pspec, pspec,
            pl.BlockSpec((BN, 1), lambda i: (i, 0)),
            pl.BlockSpec((BN, 1), lambda i: (i, 0)),
            pl.BlockSpec((4, H, H), lambda i: (0, 0, 0)),
            pl.BlockSpec((1, H), lambda i: (0, 0)),
            pl.BlockSpec((H, 1), lambda i: (0, 0)),
        ],
        out_specs=pl.BlockSpec((1, G), lambda i: (0, 0)),
        out_shape=jax.ShapeDtypeStruct((1, G), jnp.float32),
    )(h, p1, p2, p3, dinv, batch2, W1, b1r, Wout)


def kernel(x, edge_index, batch, W0, b0, W1, b1, Wout, bout):
    row = edge_index[0].astype(jnp.int32)
    col = edge_index[1].astype(jnp.int32)
    rows_p = jnp.concatenate([row, jnp.zeros((EP - E,), jnp.int32)])
    cols_p = jnp.concatenate([col, jnp.full((EP - E,), N, jnp.int32)])
    rows4d = rows_p.reshape(1, NS, NB, B)
    offs1 = (jnp.arange(C1, dtype=jnp.int32) * NP).reshape(C1, 1, 1, 1)
    offs2 = (jnp.arange(C2, dtype=jnp.int32) * NP).reshape(C2, 1, 1, 1)
    rows_c1 = rows4d + offs1
    rows_c2 = rows4d + offs2
    cols2d = cols_p.reshape(NS, NB, B)
    ones_src = jnp.ones((B, F), jnp.float32)
    zerosf = jnp.zeros((NP, F), jnp.float32)

    degacc = _sc_degree(cols2d, ones_src, zerosf)
    dinv, d2, u0 = _prep_call(degacc, x)

    p1 = _sc_prop(C1, u0.reshape(C1 * NP, F), rows_c1, cols2d, zerosf)
    u1 = _scale_call(C1, p1, d2)
    p2 = _sc_prop(C1, u1.reshape(C1 * NP, F), rows_c1, cols2d, zerosf)
    u2 = _scale_call(C1, p2, d2)
    p3 = _sc_prop(C1, u2.reshape(C1 * NP, F), rows_c1, cols2d, zerosf)

    out1, u0b = _layer1_call(x, p1, p2, p3, dinv, W0, b0.reshape(1, H))

    q1 = _sc_prop(C2, u0b.reshape(C2 * NP, F), rows_c2, cols2d, zerosf)
    v1 = _scale_call(C2, q1, d2)
    q2 = _sc_prop(C2, v1.reshape(C2 * NP, F), rows_c2, cols2d, zerosf)
    v2 = _scale_call(C2, q2, d2)
    q3 = _sc_prop(C2, v2.reshape(C2 * NP, F), rows_c2, cols2d, zerosf)

    pooled = _layer2_call(out1, q1, q2, q3, dinv, batch.reshape(N, 1),
                          W1, b1.reshape(1, H), Wout)
    return pooled.reshape(G, 1) + bout


# speedup vs baseline: 1.0050x; 1.0050x over previous
import functools

import jax
import jax.numpy as jnp
from jax import lax
from jax.experimental import pallas as pl
from jax.experimental.pallas import tpu as pltpu
from jax.experimental.pallas import tpu_sc as plsc

N = 10000
NP = 10112
E = 160000
B = 128
NB = 80
ET = NB * B
EP = 16 * ET
D = 256
H = 512
F = 128
C1 = D // F
C2 = H // F
G = 64
NC, NS = 2, 16
RPT = NP // NS
BN = 1000
GRID = N // BN


def _sc_mesh():
    return plsc.VectorSubcoreMesh(core_axis_name="c", subcore_axis_name="s")


def _sc_degree(cols2d, ones_src, zerosf):
    @functools.partial(
        pl.kernel,
        mesh=_sc_mesh(),
        out_type=jax.ShapeDtypeStruct((NC, NP, F), jnp.float32),
        scratch_types=[
            pltpu.VMEM((NB, B), jnp.int32),
            pltpu.VMEM((B, F), jnp.float32),
            pltpu.VMEM_SHARED((NP, F), jnp.float32),
        ],
    )
    def k(cols_hbm, ones_hbm, z_hbm, out_hbm, cols_v, ones_v, acc):
        c = lax.axis_index("c")
        s = lax.axis_index("s")
        pltpu.sync_copy(cols_hbm.at[s], cols_v)
        pltpu.sync_copy(ones_hbm, ones_v)
        pltpu.sync_copy(z_hbm.at[pl.ds(s * RPT, RPT)], acc.at[pl.ds(s * RPT, RPT)])
        plsc.subcore_barrier()

        def body(j, carry):
            pltpu.sync_copy(ones_v, acc.at[cols_v.at[j]], add=True)
            return carry

        lax.fori_loop(c * (NB // 2), (c + 1) * (NB // 2), body, 0)
        plsc.subcore_barrier()
        pltpu.sync_copy(acc.at[pl.ds(s * RPT, RPT)], out_hbm.at[c, pl.ds(s * RPT, RPT)])

    return k(cols2d, ones_src, zerosf)


def _sc_prop(C, u_flat, rows16, cols2d, zerosf):
    cpc = C // NC

    @functools.partial(
        pl.kernel,
        mesh=_sc_mesh(),
        out_type=jax.ShapeDtypeStruct((C, NP, F), jnp.float32),
        scratch_types=[
            pltpu.VMEM((ET,), jnp.int32),
            pltpu.VMEM((NB, B), jnp.int32),
            pltpu.VMEM((B,), jnp.int32),
            pltpu.VMEM((B, F), jnp.float32),
            pltpu.VMEM_SHARED((NP, F), jnp.float32),
            pltpu.SemaphoreType.DMA,
        ],
    )
    def k(u_hbm, rows_hbm, cols_hbm, z_hbm, out_hbm, rows_v, cols_v, gidx, gbuf, acc, sem):
        c = lax.axis_index("c")
        s = lax.axis_index("s")
        pltpu.sync_copy(rows_hbm.at[s], rows_v)
        pltpu.sync_copy(cols_hbm.at[s], cols_v)
        for ci in range(cpc):
            q = c + NC * ci
            off = q * NP
            pltpu.sync_copy(z_hbm.at[pl.ds(s * RPT, RPT)], acc.at[pl.ds(s * RPT, RPT)])
            plsc.subcore_barrier()

            def body(j, carry):
                for jj in range(B // 16):
                    rv = rows_v[pl.ds(j * B + jj * 16, 16)]
                    gidx[pl.ds(jj * 16, 16)] = rv + off
                pltpu.async_copy(u_hbm.at[gidx], gbuf, sem).wait()
                pltpu.sync_copy(gbuf, acc.at[cols_v.at[j]], add=True)
                return carry

            lax.fori_loop(0, NB, body, 0)
            plsc.subcore_barrier()
            pltpu.sync_copy(acc.at[pl.ds(s * RPT, RPT)], out_hbm.at[q, pl.ds(s * RPT, RPT)])

    return k(u_flat, rows16, cols2d, zerosf)


def _prelu(v):
    return jnp.where(v >= 0, v, 0.25 * v)


def _prep_call(degacc, x):
    def body(deg_ref, x_ref, dinv_ref, d2_ref, u0_ref):
        deg = deg_ref[0, :, 0:1] + deg_ref[1, :, 0:1]
        dinv = jnp.where(deg > 0, lax.rsqrt(deg), 0.0)
        dinv_ref[...] = dinv
        d2_ref[...] = dinv * dinv
        for q in range(C1):
            u0_ref[q] = x_ref[:, q * F:(q + 1) * F] * dinv

    return pl.pallas_call(
        body,
        grid=(GRID,),
        in_specs=[
            pl.BlockSpec((NC, BN, F), lambda i: (0, i, 0)),
            pl.BlockSpec((BN, D), lambda i: (i, 0)),
        ],
        out_specs=[
            pl.BlockSpec((BN, 1), lambda i: (i, 0)),
            pl.BlockSpec((BN, 1), lambda i: (i, 0)),
            pl.BlockSpec((C1, BN, F), lambda i: (0, i, 0)),
        ],
        out_shape=[
            jax.ShapeDtypeStruct((N, 1), jnp.float32),
            jax.ShapeDtypeStruct((N, 1), jnp.float32),
            jax.ShapeDtypeStruct((C1, NP, F), jnp.float32),
        ],
    )(degacc, x)


def _scale_call(C, p, d2):
    def body(p_ref, d2_ref, u_ref):
        u_ref[0] = p_ref[0] * d2_ref[...]

    return pl.pallas_call(
        body,
        grid=(C, GRID),
        in_specs=[
            pl.BlockSpec((1, BN, F), lambda c, i: (c, i, 0)),
            pl.BlockSpec((BN, 1), lambda c, i: (i, 0)),
        ],
        out_specs=pl.BlockSpec((1, BN, F), lambda c, i: (c, i, 0)),
        out_shape=jax.ShapeDtypeStruct((C, NP, F), jnp.float32),
    )(p, d2)


def _layer1_call(x, p1, p2, p3, dinv, W0, b0r):
    def body(x_ref, p1_ref, p2_ref, p3_ref, dinv_ref, w_ref, b_ref, out_ref, u_ref):
        dinv = dinv_ref[...]
        acc = jnp.dot(x_ref[...], w_ref[0], preferred_element_type=jnp.float32)
        for k, p_ref in ((1, p1_ref), (2, p2_ref), (3, p3_ref)):
            for q in range(C1):
                acc += jnp.dot(p_ref[q] * dinv, w_ref[k, q * F:(q + 1) * F, :],
                               preferred_element_type=jnp.float32)
        o = _prelu(acc + b_ref[...])
        out_ref[...] = o
        for q in range(C2):
            u_ref[q] = o[:, q * F:(q + 1) * F] * dinv

    pspec = pl.BlockSpec((C1, BN, F), lambda i: (0, i, 0))
    return pl.pallas_call(
        body,
        grid=(GRID,),
        in_specs=[
            pl.BlockSpec((BN, D), lambda i: (i, 0)),
            pspec, pspec, pspec,
            pl.BlockSpec((BN, 1), lambda i: (i, 0)),
            pl.BlockSpec((4, D, H), lambda i: (0, 0, 0)),
            pl.BlockSpec((1, H), lambda i: (0, 0)),
        ],
        out_specs=[
            pl.BlockSpec((BN, H), lambda i: (i, 0)),
            pl.BlockSpec((C2, BN, F), lambda i: (0, i, 0)),
        ],
        out_shape=[
            jax.ShapeDtypeStruct((N, H), jnp.float32),
            jax.ShapeDtypeStruct((C2, NP, F), jnp.float32),
        ],
    )(x, p1, p2, p3, dinv, W0, b0r)


def _layer2_call(h, p1, p2, p3, dinv, batch2, W1, b1r, Wout):
    def body(h_ref, p1_ref, p2_ref, p3_ref, dinv_ref, b_ref, w_ref, bias_ref,
             wout_ref, out_ref):
        i = pl.program_id(0)
        dinv = dinv_ref[...]
        acc = jnp.dot(h_ref[...], w_ref[0], preferred_element_type=jnp.float32)
        for k, p_ref in ((1, p1_ref), (2, p2_ref), (3, p3_ref)):
            for q in range(C2):
                acc += jnp.dot(p_ref[q] * dinv, w_ref[k, q * F:(q + 1) * F, :],
                               preferred_element_type=jnp.float32)
        o = _prelu(acc + bias_ref[...])
        z = jnp.dot(o, wout_ref[...], preferred_element_type=jnp.float32)
        gids = lax.broadcasted_iota(jnp.int32, (BN, G), 1)
        mask = (b_ref[...] == gids).astype(jnp.float32)
        pool = jnp.sum(mask * z, axis=0, keepdims=True)

        @pl.when(i == 0)
        def _():
            out_ref[...] = jnp.zeros_like(out_ref)

        out_ref[...] += pool

    pspec = pl.BlockSpec((C2, BN, F), lambda i: (0, i, 0))
    return pl.pallas_call(
        body,
        grid=(GRID,),
        in_specs=[
            pl.BlockSpec((BN, H), lambda i: (i, 0)),
            pspec, pspec, pspec,
            pl.BlockSpec((BN, 1), lambda i: (i, 0)),
            pl.BlockSpec((BN, 1), lambda i: (i, 0)),
            pl.BlockSpec((4, H, H), lambda i: (0, 0, 0)),
            pl.BlockSpec((1, H), lambda i: (0, 0)),
            pl.BlockSpec((H, 1), lambda i: (0, 0)),
        ],
        out_specs=pl.BlockSpec((1, G), lambda i: (0, 0)),
        out_shape=jax.ShapeDtypeStruct((1, G), jnp.float32),
    )(h, p1, p2, p3, dinv, batch2, W1, b1r, Wout)


def kernel(x, edge_index, batch, W0, b0, W1, b1, Wout, bout):
    row = edge_index[0].astype(jnp.int32)
    col = edge_index[1].astype(jnp.int32)
    rows_p = jnp.concatenate([row, jnp.zeros((EP - E,), jnp.int32)])
    cols_p = jnp.concatenate([col, jnp.full((EP - E,), N, jnp.int32)])
    rows16 = rows_p.reshape(NS, ET)
    cols2d = cols_p.reshape(NS, NB, B)
    ones_src = jnp.ones((B, F), jnp.float32)
    zerosf = jnp.zeros((NP, F), jnp.float32)

    degacc = _sc_degree(cols2d, ones_src, zerosf)
    dinv, d2, u0 = _prep_call(degacc, x)

    p1 = _sc_prop(C1, u0.reshape(C1 * NP, F), rows16, cols2d, zerosf)
    u1 = _scale_call(C1, p1, d2)
    p2 = _sc_prop(C1, u1.reshape(C1 * NP, F), rows16, cols2d, zerosf)
    u2 = _scale_call(C1, p2, d2)
    p3 = _sc_prop(C1, u2.reshape(C1 * NP, F), rows16, cols2d, zerosf)

    out1, u0b = _layer1_call(x, p1, p2, p3, dinv, W0, b0.reshape(1, H))

    q1 = _sc_prop(C2, u0b.reshape(C2 * NP, F), rows16, cols2d, zerosf)
    v1 = _scale_call(C2, q1, d2)
    q2 = _sc_prop(C2, v1.reshape(C2 * NP, F), rows16, cols2d, zerosf)
    v2 = _scale_call(C2, q2, d2)
    q3 = _sc_prop(C2, v2.reshape(C2 * NP, F), rows16, cols2d, zerosf)

    pooled = _layer2_call(out1, q1, q2, q3, dinv, batch.reshape(N, 1),
                          W1, b1.reshape(1, H), Wout)
    return pooled.reshape(G, 1) + bout


# speedup vs baseline: 1.3766x; 1.3698x over previous
import functools

import jax
import jax.numpy as jnp
from jax import lax
from jax.experimental import pallas as pl
from jax.experimental.pallas import tpu as pltpu
from jax.experimental.pallas import tpu_sc as plsc

N = 10000
NP = 10112
E = 160000
B = 128
NB = 79
ET = NB * B
EP = 16 * ET
D = 256
H = 512
F = 128
C1 = D // F
C2 = H // F
G = 64
NC, NS = 2, 16
RPT = NP // NS
BN = 1000
GRID = N // BN


def _sc_mesh():
    return plsc.VectorSubcoreMesh(core_axis_name="c", subcore_axis_name="s")


def _sc_degree(cols2d, ones_src, zerosf):
    @functools.partial(
        pl.kernel,
        mesh=_sc_mesh(),
        out_type=jax.ShapeDtypeStruct((NC, NP, F), jnp.float32),
        scratch_types=[
            pltpu.VMEM((NB, B), jnp.int32),
            pltpu.VMEM((B, F), jnp.float32),
            pltpu.VMEM_SHARED((NP, F), jnp.float32),
        ],
    )
    def k(cols_hbm, ones_hbm, z_hbm, out_hbm, cols_v, ones_v, acc):
        c = lax.axis_index("c")
        s = lax.axis_index("s")
        pltpu.sync_copy(cols_hbm.at[s], cols_v)
        pltpu.sync_copy(ones_hbm, ones_v)
        pltpu.sync_copy(z_hbm.at[pl.ds(s * RPT, RPT)], acc.at[pl.ds(s * RPT, RPT)])
        plsc.subcore_barrier()
        lo = c * 40
        hi = 40 + 39 * c

        def body(j, carry):
            pltpu.sync_copy(ones_v, acc.at[cols_v.at[j]], add=True)
            return carry

        lax.fori_loop(lo, hi, body, 0)
        plsc.subcore_barrier()
        pltpu.sync_copy(acc.at[pl.ds(s * RPT, RPT)], out_hbm.at[c, pl.ds(s * RPT, RPT)])

    return k(cols2d, ones_src, zerosf)


def _sc_prop(C, u_flat, rows16, cols2d, zerosf):
    cpc = C // NC

    @functools.partial(
        pl.kernel,
        mesh=_sc_mesh(),
        out_type=jax.ShapeDtypeStruct((C, NP, F), jnp.float32),
        scratch_types=[
            pltpu.VMEM((ET,), jnp.int32),
            pltpu.VMEM((NB, B), jnp.int32),
            pltpu.VMEM((B,), jnp.int32),
            pltpu.VMEM((B, F), jnp.float32),
            pltpu.VMEM_SHARED((NP, F), jnp.float32),
            pltpu.SemaphoreType.DMA,
        ],
    )
    def k(u_hbm, rows_hbm, cols_hbm, z_hbm, out_hbm, rows_v, cols_v, gidx, gbuf, acc, sem):
        c = lax.axis_index("c")
        s = lax.axis_index("s")
        pltpu.sync_copy(rows_hbm.at[s], rows_v)
        pltpu.sync_copy(cols_hbm.at[s], cols_v)
        for ci in range(cpc):
            q = c + NC * ci
            off = q * N
            pltpu.sync_copy(z_hbm.at[pl.ds(s * RPT, RPT)], acc.at[pl.ds(s * RPT, RPT)])
            plsc.subcore_barrier()

            def body(j, carry):
                for jj in range(B // 16):
                    rv = rows_v[pl.ds(j * B + jj * 16, 16)]
                    gidx[pl.ds(jj * 16, 16)] = rv + off
                pltpu.async_copy(u_hbm.at[gidx], gbuf, sem).wait()
                pltpu.sync_copy(gbuf, acc.at[cols_v.at[j]], add=True)
                return carry

            lax.fori_loop(0, NB, body, 0)
            plsc.subcore_barrier()
            pltpu.sync_copy(acc.at[pl.ds(s * RPT, RPT)], out_hbm.at[q, pl.ds(s * RPT, RPT)])

    return k(u_flat, rows16, cols2d, zerosf)


def _prelu(v):
    return jnp.where(v >= 0, v, 0.25 * v)


def _prep_call(deg2, x):
    def body(deg_ref, x_ref, dinv_ref, d2_ref, u0_ref):
        deg = deg_ref[0] + deg_ref[1]
        dinv = jnp.where(deg > 0, lax.rsqrt(deg), 0.0)
        dinv_ref[...] = dinv
        d2_ref[...] = dinv * dinv
        for q in range(C1):
            u0_ref[q] = x_ref[:, q * F:(q + 1) * F] * dinv

    return pl.pallas_call(
        body,
        grid=(GRID,),
        in_specs=[
            pl.BlockSpec((NC, BN, 1), lambda i: (0, i, 0)),
            pl.BlockSpec((BN, D), lambda i: (i, 0)),
        ],
        out_specs=[
            pl.BlockSpec((BN, 1), lambda i: (i, 0)),
            pl.BlockSpec((BN, 1), lambda i: (i, 0)),
            pl.BlockSpec((C1, BN, F), lambda i: (0, i, 0)),
        ],
        out_shape=[
            jax.ShapeDtypeStruct((N, 1), jnp.float32),
            jax.ShapeDtypeStruct((N, 1), jnp.float32),
            jax.ShapeDtypeStruct((C1, N, F), jnp.float32),
        ],
    )(deg2, x)


def _scale_call(C, p, d2):
    def body(p_ref, d2_ref, u_ref):
        u_ref[0] = p_ref[0] * d2_ref[...]

    return pl.pallas_call(
        body,
        grid=(C, GRID),
        in_specs=[
            pl.BlockSpec((1, BN, F), lambda c, i: (c, i, 0)),
            pl.BlockSpec((BN, 1), lambda c, i: (i, 0)),
        ],
        out_specs=pl.BlockSpec((1, BN, F), lambda c, i: (c, i, 0)),
        out_shape=jax.ShapeDtypeStruct((C, N, F), jnp.float32),
    )(p, d2)


def _layer1_call(x, p1, p2, p3, dinv, W0, b0r):
    def body(x_ref, p1_ref, p2_ref, p3_ref, dinv_ref, w_ref, b_ref, out_ref, u_ref):
        dinv = dinv_ref[...]
        acc = jnp.dot(x_ref[...], w_ref[0], preferred_element_type=jnp.float32)
        for k, p_ref in ((1, p1_ref), (2, p2_ref), (3, p3_ref)):
            for q in range(C1):
                acc += jnp.dot(p_ref[q] * dinv, w_ref[k, q * F:(q + 1) * F, :],
                               preferred_element_type=jnp.float32)
        o = _prelu(acc + b_ref[...])
        out_ref[...] = o
        for q in range(C2):
            u_ref[q] = o[:, q * F:(q + 1) * F] * dinv

    pspec = pl.BlockSpec((C1, BN, F), lambda i: (0, i, 0))
    return pl.pallas_call(
        body,
        grid=(GRID,),
        in_specs=[
            pl.BlockSpec((BN, D), lambda i: (i, 0)),
            pspec, pspec, pspec,
            pl.BlockSpec((BN, 1), lambda i: (i, 0)),
            pl.BlockSpec((4, D, H), lambda i: (0, 0, 0)),
            pl.BlockSpec((1, H), lambda i: (0, 0)),
        ],
        out_specs=[
            pl.BlockSpec((BN, H), lambda i: (i, 0)),
            pl.BlockSpec((C2, BN, F), lambda i: (0, i, 0)),
        ],
        out_shape=[
            jax.ShapeDtypeStruct((N, H), jnp.float32),
            jax.ShapeDtypeStruct((C2, N, F), jnp.float32),
        ],
    )(x, p1, p2, p3, dinv, W0, b0r)


def _layer2_call(h, p1, p2, p3, dinv, batch2, W1, b1r, Wout):
    def body(h_ref, p1_ref, p2_ref, p3_ref, dinv_ref, b_ref, w_ref, bias_ref,
             wout_ref, out_ref):
        i = pl.program_id(0)
        dinv = dinv_ref[...]
        acc = jnp.dot(h_ref[...], w_ref[0], preferred_element_type=jnp.float32)
        for k, p_ref in ((1, p1_ref), (2, p2_ref), (3, p3_ref)):
            for q in range(C2):
                acc += jnp.dot(p_ref[q] * dinv, w_ref[k, q * F:(q + 1) * F, :],
                               preferred_element_type=jnp.float32)
        o = _prelu(acc + bias_ref[...])
        z = jnp.dot(o, wout_ref[...], preferred_element_type=jnp.float32)
        gids = lax.broadcasted_iota(jnp.int32, (BN, G), 1)
        mask = (b_ref[...] == gids).astype(jnp.float32)
        pool = jnp.sum(mask * z, axis=0, keepdims=True)

        @pl.when(i == 0)
        def _():
            out_ref[...] = jnp.zeros_like(out_ref)

        out_ref[...] += pool

    pspec = pl.BlockSpec((C2, BN, F), lambda i: (0, i, 0))
    return pl.pallas_call(
        body,
        grid=(GRID,),
        in_specs=[
            pl.BlockSpec((BN, H), lambda i: (i, 0)),
            pspec, pspec, pspec,
            pl.BlockSpec((BN, 1), lambda i: (i, 0)),
            pl.BlockSpec((BN, 1), lambda i: (i, 0)),
            pl.BlockSpec((4, H, H), lambda i: (0, 0, 0)),
            pl.BlockSpec((1, H), lambda i: (0, 0)),
            pl.BlockSpec((H, 1), lambda i: (0, 0)),
        ],
        out_specs=pl.BlockSpec((1, G), lambda i: (0, 0)),
        out_shape=jax.ShapeDtypeStruct((1, G), jnp.float32),
    )(h, p1, p2, p3, dinv, batch2, W1, b1r, Wout)


def kernel(x, edge_index, batch, W0, b0, W1, b1, Wout, bout):
    row = edge_index[0].astype(jnp.int32)
    col = edge_index[1].astype(jnp.int32)
    rows_p = jnp.concatenate([row, jnp.zeros((EP - E,), jnp.int32)])
    cols_p = jnp.concatenate([col, jnp.full((EP - E,), N, jnp.int32)])
    rows16 = rows_p.reshape(NS, ET)
    cols2d = cols_p.reshape(NS, NB, B)
    ones_src = jnp.ones((B, F), jnp.float32)
    zerosf = jnp.zeros((NP, F), jnp.float32)

    degacc = _sc_degree(cols2d, ones_src, zerosf)
    deg2 = degacc[:, :N, 0:1]
    dinv, d2, u0 = _prep_call(deg2, x)

    p1 = _sc_prop(C1, u0.reshape(C1 * N, F), rows16, cols2d, zerosf)[:, :N]
    u1 = _scale_call(C1, p1, d2)
    p2 = _sc_prop(C1, u1.reshape(C1 * N, F), rows16, cols2d, zerosf)[:, :N]
    u2 = _scale_call(C1, p2, d2)
    p3 = _sc_prop(C1, u2.reshape(C1 * N, F), rows16, cols2d, zerosf)[:, :N]

    out1, u0b = _layer1_call(x, p1, p2, p3, dinv, W0, b0.reshape(1, H))

    q1 = _sc_prop(C2, u0b.reshape(C2 * N, F), rows16, cols2d, zerosf)[:, :N]
    v1 = _scale_call(C2, q1, d2)
    q2 = _sc_prop(C2, v1.reshape(C2 * N, F), rows16, cols2d, zerosf)[:, :N]
    v2 = _scale_call(C2, q2, d2)
    q3 = _sc_prop(C2, v2.reshape(C2 * N, F), rows16, cols2d, zerosf)[:, :N]

    pooled = _layer2_call(out1, q1, q2, q3, dinv, batch.reshape(N, 1),
                          W1, b1.reshape(1, H), Wout)
    return pooled.reshape(G, 1) + bout
